# Initial kernel scaffold; baseline (speedup 1.0000x reference)
import jax
import jax.numpy as jnp
from jax.experimental import pallas as pl


def kernel(edge_index, edge_weight, user_weight, item_weight):
    raise NotImplementedError("write your pallas kernel here")


# speedup vs baseline: 5.4330x; 5.4330x over previous
import functools

import jax
import jax.numpy as jnp
import numpy as np
from jax import lax
from jax.experimental import pallas as pl
from jax.experimental.pallas import tpu as pltpu
from jax.experimental.pallas import tpu_sc as plsc

N_USERS = 25000
N_NODES = 50000
D = 64
H = 32
E = 800000
EPS = 0.1

NC, NS, L = 2, 16, 16
NP = 50048
W = 512
SUB = 128
NSUB = W // SUB
NWIN = 98
EPT = NWIN * W
E_PAD = EPT * NS
ROWS_PT = NP // NS
OWIN = 136
NOWIN = ROWS_PT // OWIN

_f32 = jnp.float32
_i32 = jnp.int32


def _body(col2, row2, wp, tbl0, nnf,
          fin_o, cl_o, e1f,
          acc, colb, rowb, wb, rowsb, gsem, ssem):
  c = lax.axis_index("c")
  s = lax.axis_index("s")
  base_t = (c * NP).astype(_i32)
  base_vec = jnp.full((L,), 0, _i32) + base_t
  nodes0 = s * ROWS_PT

  zvec = jnp.zeros((L,), _f32)

  def zero_acc():
    @pl.loop(0, OWIN)
    def _zero_zb(i):
      rowsb[i, pl.ds(0, L)] = zvec
      rowsb[i, pl.ds(L, L)] = zvec

    for k in range(NOWIN):
      pltpu.sync_copy(rowsb.at[pl.ds(0, OWIN)],
                      acc.at[pl.ds(nodes0 + k * OWIN, OWIN)])

  def run_layer(tbl_ref):
    @pl.loop(0, NWIN)
    def _win(g):
      er = s * (EPT // SUB) + g * NSUB
      eo = s * EPT + g * W
      pltpu.sync_copy(col2.at[pl.ds(er, NSUB)], colb)
      pltpu.sync_copy(row2.at[pl.ds(er, NSUB)], rowb)
      pltpu.sync_copy(wp.at[pl.ds(eo, W)], wb)

      @pl.loop(0, NSUB)
      def _fix(i):
        for k in range(SUB // L):
          sl = pl.ds(k * L, L)
          colb[i, sl] = colb[i, sl] + base_vec

      gds = [
          pltpu.async_copy(tbl_ref.at[colb.at[j]],
                           rowsb.at[pl.ds(j * SUB, SUB)], gsem)
          for j in range(NSUB)
      ]
      for d_ in gds:
        d_.wait()

      @pl.loop(0, W // L)
      def _mul(grp):
        e0i = grp * L
        wv16 = wb[pl.ds(e0i, L)]
        for j2 in range(L):
          el = e0i + j2
          wv = jnp.full((L,), 0, _f32) + wv16[j2]
          rowsb[el, pl.ds(0, L)] = rowsb[el, pl.ds(0, L)] * wv
          rowsb[el, pl.ds(L, L)] = rowsb[el, pl.ds(L, L)] * wv

      sds = [
          pltpu.async_copy(rowsb.at[pl.ds(j * SUB, SUB)],
                           acc.at[rowb.at[j]], ssem, add=True)
          for j in range(NSUB)
      ]
      for d_ in sds:
        d_.wait()

  zero_acc()
  plsc.subcore_barrier()
  run_layer(tbl0)
  plsc.subcore_barrier()
  pltpu.sync_copy(acc.at[pl.ds(nodes0, ROWS_PT)],
                  e1f.at[pl.ds(base_t + nodes0, ROWS_PT)])
  zero_acc()
  plsc.subcore_barrier()
  run_layer(e1f)
  plsc.subcore_barrier()

  for k in range(NOWIN):
    r0 = nodes0 + k * OWIN
    pltpu.sync_copy(e1f.at[pl.ds(base_t + r0, OWIN)], rowsb.at[pl.ds(0, OWIN)])
    pltpu.sync_copy(acc.at[pl.ds(r0, OWIN)], rowsb.at[pl.ds(OWIN, OWIN)])
    pltpu.sync_copy(nnf.at[pl.ds(base_t + r0, OWIN)],
                    rowsb.at[pl.ds(2 * OWIN, OWIN)])

    @pl.loop(0, OWIN)
    def _ep(i):
      for h in range(2):
        sl = pl.ds(h * L, L)
        e1v = rowsb[i, sl]
        e2v = rowsb[OWIN + i, sl]
        nv = rowsb[2 * OWIN + i, sl]
        rowsb[i, sl] = (e1v + e2v) * 0.5
        rowsb[OWIN + i, sl] = e2v + jnp.sign(e2v) * nv

    pltpu.sync_copy(rowsb.at[pl.ds(0, OWIN)],
                    fin_o.at[pl.ds(base_t + r0, OWIN)])
    pltpu.sync_copy(rowsb.at[pl.ds(OWIN, OWIN)],
                    cl_o.at[pl.ds(base_t + r0, OWIN)])


@functools.partial(
    pl.kernel,
    out_type=(
        jax.ShapeDtypeStruct((NC * NP, H), _f32),
        jax.ShapeDtypeStruct((NC * NP, H), _f32),
        jax.ShapeDtypeStruct((NC * NP, H), _f32),
    ),
    mesh=plsc.VectorSubcoreMesh(
        core_axis_name="c", subcore_axis_name="s", num_cores=NC,
        num_subcores=NS),
    compiler_params=pltpu.CompilerParams(use_tc_tiling_on_sc=False),
    scratch_types=(
        pltpu.VMEM_SHARED((NP, H), _f32),
        pltpu.VMEM((NSUB, SUB), _i32),
        pltpu.VMEM((NSUB, SUB), _i32),
        pltpu.VMEM((W,), _f32),
        pltpu.VMEM((W, H), _f32),
        pltpu.SemaphoreType.DMA,
        pltpu.SemaphoreType.DMA,
    ),
)
def _sc_propagate(*args):
  _body(*args)


def kernel(edge_index, edge_weight, user_weight, item_weight):
  e0 = jnp.concatenate([user_weight, item_weight], axis=0)
  zp = jnp.zeros((NP - N_NODES, H), _f32)
  tbl0 = jnp.concatenate([e0[:, :H], zp, e0[:, H:], zp], axis=0)

  noise = jax.random.uniform(
      jax.random.fold_in(jax.random.key(42), 1), (N_NODES, D), dtype=_f32)
  nrm = jnp.maximum(jnp.linalg.norm(noise, axis=-1, keepdims=True), 1e-12)
  nn = noise / nrm * EPS
  nnf = jnp.concatenate([nn[:, :H], zp, nn[:, H:], zp], axis=0)

  row = edge_index[0]
  col = edge_index[1]
  pad = E_PAD - E
  padidx = (np.arange(pad) % N_NODES).astype(np.int32)
  colp = jnp.concatenate([col, jnp.asarray(padidx)])
  rowp = jnp.concatenate([row, jnp.asarray(padidx)])
  wp = jnp.concatenate([edge_weight, jnp.zeros((pad,), _f32)])
  col2 = colp.reshape(E_PAD // SUB, SUB)
  row2 = rowp.reshape(E_PAD // SUB, SUB)

  fin_h, cl_h, _ = _sc_propagate(col2, row2, wp, tbl0, nnf)

  fin = jnp.concatenate([fin_h[:N_NODES], fin_h[NP:NP + N_NODES]], axis=1)
  cl = jnp.concatenate([cl_h[:N_NODES], cl_h[NP:NP + N_NODES]], axis=1)
  return (fin[:N_USERS], fin[N_USERS:], cl[:N_USERS], cl[N_USERS:])
